# baseline (speedup 1.0000x reference)
import jax
import jax.numpy as jnp
from jax.experimental import pallas as pl

_F = 100
_D = 64
_RB = 256


def _fe_kernel(mt_ref, xt_ref, o_ref):
    i = pl.program_id(0)
    x8 = xt_ref[...]
    ones = jnp.ones((1, x8.shape[1]), dtype=jnp.bfloat16)
    sub = i % 2
    for s in range(2):
        @pl.when(sub == s)
        def _(s=s):
            xa = jnp.concatenate(
                [x8[4 * s:4 * s + 4].astype(jnp.bfloat16), ones], axis=0
            )
            o_ref[...] = jax.lax.dot_general(
                mt_ref[...], xa,
                (((0,), (0,)), ((), ())),
                preferred_element_type=jnp.float32,
            )


def kernel(x, emb_table, w, b):
    B, F = x.shape
    D = emb_table.shape[1]
    FD = F * D
    xt = x.T
    wt = jnp.broadcast_to(w.reshape(1, D), (F, D)).reshape(1, FD)
    tb = (emb_table + b[None, :]).reshape(1, FD)
    lane = jax.lax.broadcasted_iota(jnp.int32, (1, FD), 1)
    g = (lane % _RB) // D
    rows = [jnp.where(g == j, wt, 0.0) for j in range(4)]
    mt = jnp.concatenate(rows + [tb], axis=0).astype(jnp.bfloat16)
    grid = (FD // _RB,)
    out_t = pl.pallas_call(
        _fe_kernel,
        grid=grid,
        in_specs=[
            pl.BlockSpec((5, _RB), lambda i: (0, i)),
            pl.BlockSpec((8, B), lambda i: (i // 2, 0)),
        ],
        out_specs=pl.BlockSpec((_RB, B), lambda i: (i, 0)),
        out_shape=jax.ShapeDtypeStruct((FD, B), jnp.float32),
    )(mt, xt)
    return out_t.reshape(F, D, B).transpose(2, 0, 1)
